# baseline (speedup 1.0000x reference)
import functools

import jax
import jax.numpy as jnp
from jax import lax
from jax.experimental import pallas as pl
from jax.experimental.pallas import tpu as pltpu
from jax.experimental.pallas import tpu_sc as plsc

NUM_TOKENS = 32768
NSRC = 32768
IN_FEAT = 128
DIM = 256
NC, NS, L = 2, 16, 16
NW = NC * NS
TOK_PER_W = NUM_TOKENS // NW
CHUNK = 128
NCHUNK = TOK_PER_W // CHUNK
IDX_BITS = 15


def _mm_body(x_ref, w_ref, b_ref, o_ref):
    o_ref[...] = (
        jnp.dot(x_ref[...], w_ref[...], preferred_element_type=jnp.float32)
        + b_ref[...]
    )


def _matmul(x, W, b):
    M = x.shape[0]
    BM = 1024
    return pl.pallas_call(
        _mm_body,
        grid=(M // BM,),
        in_specs=[
            pl.BlockSpec((BM, IN_FEAT), lambda i: (i, 0)),
            pl.BlockSpec((IN_FEAT, DIM), lambda i: (0, 0)),
            pl.BlockSpec((1, DIM), lambda i: (0, 0)),
        ],
        out_specs=pl.BlockSpec((BM, DIM), lambda i: (i, 0)),
        out_shape=jax.ShapeDtypeStruct((M, DIM), jnp.float32),
    )(x, W, b.reshape(1, DIM))


_mesh = plsc.VectorSubcoreMesh(core_axis_name="c", subcore_axis_name="s")


@functools.partial(
    pl.kernel,
    out_type=jax.ShapeDtypeStruct((NUM_TOKENS, DIM), jnp.float32),
    mesh=_mesh,
    compiler_params=pltpu.CompilerParams(needs_layout_passes=False),
    scratch_types=[
        pltpu.VMEM((NSRC,), jnp.int32),
        pltpu.VMEM((TOK_PER_W,), jnp.int32),
        pltpu.VMEM((TOK_PER_W,), jnp.int32),
        pltpu.VMEM((TOK_PER_W,), jnp.int32),
        pltpu.VMEM((TOK_PER_W + L,), jnp.float32),
        pltpu.VMEM((CHUNK, DIM), jnp.float32),
        pltpu.VMEM((CHUNK, DIM), jnp.float32),
        pltpu.SemaphoreType.DMA,
        pltpu.SemaphoreType.DMA,
    ],
)
def _sc_scatter(sidx_hbm, peidx_hbm, x_hbm, pe_hbm, out_hbm,
                idx_v, winner_v, wc_v, pw_v, valid_v, xrows_v, perows_v,
                sem1, sem2):
    wid = lax.axis_index("s") * NC + lax.axis_index("c")
    base = wid * TOK_PER_W

    pltpu.sync_copy(sidx_hbm, idx_v)
    neg1 = jnp.full((L,), -1, jnp.int32)

    def init_body(g, _):
        winner_v[pl.ds(g * L, L)] = neg1
        return 0

    lax.fori_loop(0, TOK_PER_W // L, init_body, 0)

    sent = jnp.int32(2**31 - 1)
    iota = lax.iota(jnp.int32, L)
    shift_idx = jnp.minimum(iota + 1, L - 1)
    last_lane = iota == (L - 1)

    def scan_body(g, _):
        idx16 = idx_v[pl.ds(g * L, L)]
        local = idx16 - base
        inr = (local >= 0) & (local < TOK_PER_W)
        i_vec = g * L + iota
        key = jnp.where(inr, (local << IDX_BITS) | i_vec, sent)
        skey, _ = plsc.sort_key_val(key, key)
        nxt = skey.at[shift_idx].get(mode="promise_in_bounds")
        tok = skey >> IDX_BITS
        keep = ((tok != (nxt >> IDX_BITS)) | last_lane) & (skey != sent)
        tok_st = tok & (TOK_PER_W - 1)
        ival = skey & (NSRC - 1)
        plsc.store_scatter(winner_v, [tok_st], ival, mask=keep)
        return 0

    lax.fori_loop(0, NSRC // L, scan_body, 0)

    pltpu.sync_copy(peidx_hbm, idx_v)

    def pw_body(g, _):
        sl = pl.ds(g * L, L)
        w16 = winner_v[sl]
        wcl = jnp.maximum(w16, 0)
        wc_v[sl] = wcl
        pw_v[sl] = plsc.load_gather(idx_v, [wcl])
        valid_v[sl] = jnp.where(w16 >= 0, 1.0, 0.0).astype(jnp.float32)
        return 0

    lax.fori_loop(0, TOK_PER_W // L, pw_body, 0)

    for c in range(NCHUNK):
        gx = pltpu.async_copy(
            x_hbm.at[wc_v.at[pl.ds(c * CHUNK, CHUNK)]], xrows_v, sem1)
        gp = pltpu.async_copy(
            pe_hbm.at[pw_v.at[pl.ds(c * CHUNK, CHUNK)]], perows_v, sem2)
        gx.wait()
        gp.wait()

        def row_body(r, _, c=c):
            fvec = valid_v[pl.ds(c * CHUNK + r, L)]
            fv = jnp.broadcast_to(fvec[0], (L,))
            for j in range(DIM // L):
                sl = pl.ds(j * L, L)
                xrows_v[r, sl] = (xrows_v[r, sl] + perows_v[r, sl]) * fv
            return 0

        lax.fori_loop(0, CHUNK, row_body, 0)
        pltpu.sync_copy(xrows_v, out_hbm.at[pl.ds(base + c * CHUNK, CHUNK)])


def kernel(sdata, scatter_idxs, pe_idxs, pe_embed, W, b):
    x = _matmul(sdata.reshape(-1, IN_FEAT), W, b)
    return _sc_scatter(
        scatter_idxs.astype(jnp.int32), pe_idxs.astype(jnp.int32), x, pe_embed)
